# Initial kernel scaffold; baseline (speedup 1.0000x reference)
import jax
import jax.numpy as jnp
from jax.experimental import pallas as pl


def kernel(x, weight_g, bias_g):
    raise NotImplementedError("write your pallas kernel here")


# speedup vs baseline: 1.0595x; 1.0595x over previous
import functools

import jax
import jax.numpy as jnp
from jax.experimental import pallas as pl
from jax.experimental.pallas import tpu as pltpu


def _group_kernel(w_ref, b_ref, x_ref, o_ref, *, inv_cnt_c, inv_cnt_g,
                  s_total, eps, bn_eps):
    x = x_ref[...].astype(jnp.float32)
    red_axes = (3, 4)
    sum_ns = jnp.sum(x, axis=red_axes, keepdims=True)
    ssq_ns = jnp.sum(x * x, axis=red_axes, keepdims=True)

    sum_c = jnp.sum(sum_ns, axis=0, keepdims=True)
    ssq_c = jnp.sum(ssq_ns, axis=0, keepdims=True)
    mean_c = sum_c * inv_cnt_c
    var_c = jnp.maximum(ssq_c * inv_cnt_c - mean_c * mean_c, 0.0)
    a_c = jax.lax.rsqrt(var_c + bn_eps)
    b_c = -mean_c * a_c

    sum_y = a_c * sum_ns + b_c * s_total
    ssq_y = (a_c * a_c) * ssq_ns + (2.0 * a_c * b_c) * sum_ns \
        + (b_c * b_c) * s_total
    mean_g = jnp.sum(sum_y, axis=2, keepdims=True) * inv_cnt_g
    var_g = jnp.maximum(
        jnp.sum(ssq_y, axis=2, keepdims=True) * inv_cnt_g - mean_g * mean_g,
        0.0)
    inv_std_g = jax.lax.rsqrt(var_g + eps)

    scale = w_ref[0, 0, 0, 0, 0] * inv_std_g
    a_row = scale * a_c
    b_row = scale * (b_c - mean_g) + b_ref[0, 0, 0, 0, 0]
    o_ref[...] = (a_row * x + b_row).astype(o_ref.dtype)


def _split_spatial(s):
    if s % 128 != 0:
        return 1, s
    chunks = s // 128
    for s1 in range(min(chunks, 512), 0, -1):
        if chunks % s1 == 0 and (s1 % 8 == 0 or s1 == 1):
            return s1, (chunks // s1) * 128
    return 1, s


def kernel(x, weight_g, bias_g):
    num_groups = 8
    eps = 1e-5
    bn_eps = 1e-5
    N, C, D, H, W = x.shape
    G = num_groups
    Cg = C // G
    S = D * H * W

    s1, s2 = _split_spatial(S)

    x5 = x.reshape(N, G, Cg, s1, s2)
    w5 = weight_g.astype(jnp.float32).reshape(1, G, 1, 1, 1)
    b5 = bias_g.astype(jnp.float32).reshape(1, G, 1, 1, 1)

    kern = functools.partial(
        _group_kernel,
        inv_cnt_c=1.0 / float(N * S),
        inv_cnt_g=1.0 / float(Cg * S),
        s_total=float(S),
        eps=float(eps),
        bn_eps=float(bn_eps))

    out5 = pl.pallas_call(
        kern,
        grid=(G,),
        in_specs=[
            pl.BlockSpec((1, 1, 1, 1, 1), lambda g: (0, g, 0, 0, 0)),
            pl.BlockSpec((1, 1, 1, 1, 1), lambda g: (0, g, 0, 0, 0)),
            pl.BlockSpec((N, 1, Cg, s1, s2), lambda g: (0, g, 0, 0, 0)),
        ],
        out_specs=pl.BlockSpec((N, 1, Cg, s1, s2), lambda g: (0, g, 0, 0, 0)),
        out_shape=jax.ShapeDtypeStruct((N, G, Cg, s1, s2), x.dtype),
        compiler_params=pltpu.CompilerParams(
            dimension_semantics=("parallel",),
            vmem_limit_bytes=56 * 1024 * 1024),
    )(w5, b5, x5)
    return out5.reshape(N, C, D, H, W)


# speedup vs baseline: 1.0757x; 1.0152x over previous
import functools

import jax
import jax.numpy as jnp
from jax.experimental import pallas as pl
from jax.experimental.pallas import tpu as pltpu


def _group_kernel(w_ref, b_ref, x_ref, o_ref, *, inv_cnt_c, inv_cnt_g,
                  s_total, eps, bn_eps):
    g = pl.program_id(0)
    x = x_ref[...].astype(jnp.float32)
    red_axes = (3, 4)
    sum_ns = jnp.sum(x, axis=red_axes, keepdims=True)
    ssq_ns = jnp.sum(x * x, axis=red_axes, keepdims=True)

    sum_c = jnp.sum(sum_ns, axis=0, keepdims=True)
    ssq_c = jnp.sum(ssq_ns, axis=0, keepdims=True)
    mean_c = sum_c * inv_cnt_c
    var_c = jnp.maximum(ssq_c * inv_cnt_c - mean_c * mean_c, 0.0)
    a_c = jax.lax.rsqrt(var_c + bn_eps)
    b_c = -mean_c * a_c

    sum_y = a_c * sum_ns + b_c * s_total
    ssq_y = (a_c * a_c) * ssq_ns + (2.0 * a_c * b_c) * sum_ns \
        + (b_c * b_c) * s_total
    mean_g = jnp.sum(sum_y, axis=2, keepdims=True) * inv_cnt_g
    var_g = jnp.maximum(
        jnp.sum(ssq_y, axis=2, keepdims=True) * inv_cnt_g - mean_g * mean_g,
        0.0)
    inv_std_g = jax.lax.rsqrt(var_g + eps)

    scale = w_ref[g] * inv_std_g
    a_row = scale * a_c
    b_row = scale * (b_c - mean_g) + b_ref[g]
    o_ref[...] = (a_row * x + b_row).astype(o_ref.dtype)


def _split_spatial(s):
    if s % 128 != 0:
        return 1, s
    chunks = s // 128
    for s1 in range(min(chunks, 512), 0, -1):
        if chunks % s1 == 0 and (s1 % 8 == 0 or s1 == 1):
            return s1, (chunks // s1) * 128
    return 1, s


def kernel(x, weight_g, bias_g):
    num_groups = 8
    eps = 1e-5
    bn_eps = 1e-5
    N, C, D, H, W = x.shape
    G = num_groups
    Cg = C // G
    S = D * H * W

    s1, s2 = _split_spatial(S)

    x5 = x.reshape(N, G, Cg, s1, s2)
    w1 = weight_g.astype(jnp.float32)
    b1 = bias_g.astype(jnp.float32)

    kern = functools.partial(
        _group_kernel,
        inv_cnt_c=1.0 / float(N * S),
        inv_cnt_g=1.0 / float(Cg * S),
        s_total=float(S),
        eps=float(eps),
        bn_eps=float(bn_eps))

    out5 = pl.pallas_call(
        kern,
        grid=(G,),
        in_specs=[
            pl.BlockSpec(memory_space=pltpu.SMEM),
            pl.BlockSpec(memory_space=pltpu.SMEM),
            pl.BlockSpec((N, 1, Cg, s1, s2), lambda g: (0, g, 0, 0, 0)),
        ],
        out_specs=pl.BlockSpec((N, 1, Cg, s1, s2), lambda g: (0, g, 0, 0, 0)),
        out_shape=jax.ShapeDtypeStruct((N, G, Cg, s1, s2), x.dtype),
        compiler_params=pltpu.CompilerParams(
            dimension_semantics=("parallel",),
            vmem_limit_bytes=56 * 1024 * 1024),
    )(w1, b1, x5)
    return out5.reshape(N, C, D, H, W)
